# Initial kernel scaffold; baseline (speedup 1.0000x reference)
import functools

import jax
import jax.numpy as jnp
from jax import lax
from jax.experimental import pallas as pl
from jax.experimental.pallas import tpu as pltpu
from jax.experimental.pallas import tpu_sc as plsc

D = 64
B, L = 4096, 200
N_TOK = B * L
NW = 32
TPW = N_TOK // NW
LANES = 16
BLK = 512
NBLK = TPW // BLK
CH = 128


def _embed_lookup(targets_flat, table):
    mesh = plsc.VectorSubcoreMesh(core_axis_name="c", subcore_axis_name="s")

    @functools.partial(
        pl.kernel,
        mesh=mesh,
        out_type=jax.ShapeDtypeStruct((N_TOK, D), jnp.float32),
        scratch_types=[
            pltpu.VMEM((TPW,), jnp.int32),
            pltpu.VMEM((TPW,), jnp.int32),
            pltpu.VMEM((2, BLK, D), jnp.float32),
            pltpu.SemaphoreType.DMA,
            pltpu.SemaphoreType.DMA,
            pltpu.SemaphoreType.DMA,
            pltpu.SemaphoreType.DMA,
        ],
    )
    def k(tgt_hbm, table_hbm, out_hbm, raw_v, idx_v, rows_v,
          gsem0, gsem1, ssem0, ssem1):
        gsem = (gsem0, gsem1)
        ssem = (ssem0, ssem1)
        wid = lax.axis_index("s") * 2 + lax.axis_index("c")
        base = wid * TPW
        pltpu.sync_copy(tgt_hbm.at[pl.ds(base, TPW)], raw_v)

        lane = lax.iota(jnp.int32, LANES)

        def compute_idx(i):
            def body(j, carry):
                p = i * BLK + j * LANES + lane
                g = plsc.load_gather(raw_v, [jnp.maximum(p - 1, 0)])
                idx_v[pl.ds(i * BLK + j * LANES, LANES)] = jnp.where(
                    lax.rem(p, L) == 0, 0, g)
                return carry
            lax.fori_loop(0, BLK // LANES, body, 0)

        def fire_gather(i, b):
            for c in range(BLK // CH):
                pltpu.async_copy(
                    table_hbm.at[idx_v.at[pl.ds(i * BLK + c * CH, CH)]],
                    rows_v.at[b, pl.ds(c * CH, CH)],
                    gsem[b])

        def wait_gather(b):
            pltpu.make_async_copy(
                out_hbm.at[pl.ds(0, BLK)], rows_v.at[b], gsem[b]).wait()

        def start_store(i, b):
            pltpu.async_copy(
                rows_v.at[b], out_hbm.at[pl.ds(base + i * BLK, BLK)], ssem[b])

        def wait_store(b):
            pltpu.make_async_copy(
                rows_v.at[b], out_hbm.at[pl.ds(0, BLK)], ssem[b]).wait()

        compute_idx(0)
        fire_gather(0, 0)

        def blk2(g, carry):
            for b in range(2):
                i = g * 2 + b
                @pl.when(i + 1 < NBLK)
                def _():
                    compute_idx(i + 1)
                wait_gather(b)
                start_store(i, b)

                @pl.when(i + 1 < NBLK)
                def _():
                    @pl.when(i >= 1)
                    def _():
                        wait_store(1 - b)
                    fire_gather(i + 1, 1 - b)
            return carry

        lax.fori_loop(0, NBLK // 2, blk2, 0)
        wait_store(0)
        wait_store(1)

    return k(targets_flat, table)


def kernel(targets, table):
    flat = targets.astype(jnp.int32).reshape(N_TOK)
    out = _embed_lookup(flat, table)
    return out.reshape(B, L, D)


# speedup vs baseline: 1.0848x; 1.0848x over previous
import functools

import jax
import jax.numpy as jnp
from jax import lax
from jax.experimental import pallas as pl
from jax.experimental.pallas import tpu as pltpu
from jax.experimental.pallas import tpu_sc as plsc

D = 64
DP = 128
B, L = 4096, 200
N_TOK = B * L
NW = 32
TPW = N_TOK // NW
SEQ_PER_W = TPW // L
SPB = 2
NBLK = SEQ_PER_W // SPB
CH = 128


def _embed_lookup(targets_flat, table_padded):
    mesh = plsc.VectorSubcoreMesh(core_axis_name="c", subcore_axis_name="s")

    @functools.partial(
        pl.kernel,
        mesh=mesh,
        compiler_params=pltpu.CompilerParams(use_tc_tiling_on_sc=False),
        out_type=jax.ShapeDtypeStruct((B, L, DP), jnp.float32),
        scratch_types=[
            pltpu.VMEM((TPW,), jnp.int32),
            pltpu.VMEM((2, SPB, L, DP), jnp.float32),
            pltpu.SemaphoreType.DMA,
            pltpu.SemaphoreType.DMA,
            pltpu.SemaphoreType.DMA,
            pltpu.SemaphoreType.DMA,
        ],
    )
    def k(tgt_hbm, table_hbm, out_hbm, raw_v, rows_v,
          gsem0, gsem1, ssem0, ssem1):
        gsem = (gsem0, gsem1)
        ssem = (ssem0, ssem1)
        wid = lax.axis_index("s") * 2 + lax.axis_index("c")
        base = wid * TPW
        pltpu.sync_copy(tgt_hbm.at[pl.ds(base, TPW)], raw_v)
        for b in range(2):
            for s in range(SPB):
                pltpu.sync_copy(table_hbm.at[pl.ds(0, 8)],
                                rows_v.at[b, s, pl.ds(0, 8)])

        def fire_gather(i, b):
            for s in range(SPB):
                tok0 = (i * SPB + s) * L
                pltpu.async_copy(
                    table_hbm.at[raw_v.at[pl.ds(tok0, CH)]],
                    rows_v.at[b, s, pl.ds(1, CH)],
                    gsem[b])
                pltpu.async_copy(
                    table_hbm.at[raw_v.at[pl.ds(tok0 + CH, L - 1 - CH)]],
                    rows_v.at[b, s, pl.ds(1 + CH, L - 1 - CH)],
                    gsem[b])

        def wait_gather(b):
            for s in range(SPB):
                pltpu.make_async_copy(
                    table_hbm.at[raw_v.at[pl.ds(s * L, CH)]],
                    rows_v.at[b, s, pl.ds(1, CH)],
                    gsem[b]).wait()
                pltpu.make_async_copy(
                    table_hbm.at[raw_v.at[pl.ds(s * L + CH, L - 1 - CH)]],
                    rows_v.at[b, s, pl.ds(1 + CH, L - 1 - CH)],
                    gsem[b]).wait()

        def start_store(i, b):
            pltpu.async_copy(
                rows_v.at[b],
                out_hbm.at[pl.ds(wid * SEQ_PER_W + i * SPB, SPB)],
                ssem[b])

        def wait_store(b):
            pltpu.make_async_copy(
                rows_v.at[b], out_hbm.at[pl.ds(0, SPB)], ssem[b]).wait()

        fire_gather(0, 0)

        def blk2(g, carry):
            for b in range(2):
                i = g * 2 + b
                wait_gather(b)
                start_store(i, b)

                @pl.when(i + 1 < NBLK)
                def _():
                    @pl.when(i >= 1)
                    def _():
                        wait_store(1 - b)
                    fire_gather(i + 1, 1 - b)
            return carry

        lax.fori_loop(0, NBLK // 2, blk2, 0)
        wait_store(0)
        wait_store(1)

    return k(targets_flat, table_padded)


def kernel(targets, table):
    flat = targets.astype(jnp.int32).reshape(N_TOK)
    table_padded = jnp.pad(table, ((0, 0), (0, DP - D)))
    out_padded = _embed_lookup(flat, table_padded)
    return out_padded[:, :, :D]
